# baseline (speedup 1.0000x reference)
import jax
import jax.numpy as jnp
from jax import lax
from jax.experimental import pallas as pl
from jax.experimental.pallas import tpu as pltpu

B_PAIRS = 512
QSIZE = 40
CSIZE = 50
MAX_SET = 50
N_PROP = 5
N_TIME = 3
D_STATE = 32
D_MSG = 64
TEMP = 0.1
SINK_ITERS = 20

PB = 16
GRID = B_PAIRS // PB
NPP = 2 * MAX_SET
EPP = 200
PAD_IDX = 3 * NPP


def _bmm(a, b, ca, cb):
    return lax.dot_general(
        a, b, (((ca,), (cb,)), ((0,), (0,))), preferred_element_type=jnp.float32)


def _mm(x, w):
    return jnp.dot(x, w, preferred_element_type=jnp.float32)


def _gsum(oh, x):
    hi = x.astype(jnp.bfloat16)
    lo = (x - hi.astype(jnp.float32)).astype(jnp.bfloat16)
    return _bmm(oh, hi, 2, 1) + _bmm(oh, lo, 2, 1)


def _body(nf_ref, ef_ref, fl_ref, tl_ref,
          Wn_r, bn_r, WeE_r, beE_r, cW1_r, cb1_r, cWa_r, cba_r,
          W2m_r, b2m_r, uWbc_r, ub1_r, uW2_r, ub2_r,
          t1W_r, t1b_r, t2W_r, t2b_r,
          out_ref):
    nrows = PB * NPP
    erows = PB * EPP
    H2 = 2 * D_MSG

    enc_n = _mm(nf_ref[...].reshape(nrows, -1), Wn_r[...]) + bn_r[...]
    E3 = (_mm(ef_ref[...].reshape(erows, -1), WeE_r[...])
          + beE_r[...]).reshape(PB, EPP, H2)

    fl = fl_ref[...]
    tl = tl_ref[...]
    i_n = lax.broadcasted_iota(jnp.int32, (PB, EPP, NPP), 2)
    GF = (fl[:, :, None] == i_n).astype(jnp.bfloat16)
    GT = (tl[:, :, None] == i_n).astype(jnp.bfloat16)
    i_r = lax.broadcasted_iota(jnp.int32, (PB, MAX_SET, 2 * MAX_SET), 1)

    def _scat_mat(t_idx, f_idx):
        return jnp.concatenate([
            (t_idx[:, None, :] == i_r).astype(jnp.bfloat16),
            (f_idx[:, None, :] == i_r).astype(jnp.bfloat16)], axis=1)

    Oq = _scat_mat(tl[:, :2 * MAX_SET], fl[:, :2 * MAX_SET])
    Oc = _scat_mat(tl[:, 2 * MAX_SET:] - MAX_SET, fl[:, 2 * MAX_SET:] - MAX_SET)

    i_v = lax.broadcasted_iota(jnp.int32, (1, NPP, 1), 1)
    nvalid = jnp.where((i_v >= QSIZE) & (i_v < MAX_SET), 0.0, 1.0)
    qvalid = (lax.broadcasted_iota(jnp.int32, (1, MAX_SET, 1), 1)
              < QSIZE).astype(jnp.float32)

    store = jnp.zeros((nrows, (N_PROP + 1) * D_STATE), jnp.float32)
    plan = jnp.zeros((PB, MAX_SET, MAX_SET), jnp.float32)

    for t in range(N_TIME):
        nf = enc_n
        cols = []
        for p in range(1, N_PROP + 1):
            inter = store[:, D_STATE * (p - 1):D_STATE * p]
            h1 = jnp.maximum(
                _mm(jnp.concatenate([nf, inter], axis=1), cW1_r[...])
                + cb1_r[...], 0.0)
            CA = _mm(h1, cWa_r[...]) + cba_r[...]
            U = CA[:, :H2].reshape(PB, NPP, H2)
            V = CA[:, H2:2 * H2].reshape(PB, NPP, H2)
            ucomb = CA[:, 2 * H2:2 * H2 + D_MSG]
            comb = CA[:, 2 * H2 + D_MSG:]
            Hh = jnp.maximum(_gsum(GF, U) + _gsum(GT, V) + E3, 0.0)
            X = _mm(Hh.reshape(erows, H2), W2m_r[...]) + b2m_r[...]
            Xp = X.reshape(PB, EPP, H2)
            Rq = _gsum(Oq, Xp[:, :2 * MAX_SET, :])
            Rc = _gsum(Oc, Xp[:, 2 * MAX_SET:, :])
            AR = jnp.concatenate([
                jnp.concatenate([Rq[:, :MAX_SET, :D_MSG],
                                 Rq[:, MAX_SET:, D_MSG:]], axis=2),
                jnp.concatenate([Rc[:, :MAX_SET, :D_MSG],
                                 Rc[:, MAX_SET:, D_MSG:]], axis=2)], axis=1)
            uh = jnp.maximum(
                _mm(AR.reshape(nrows, H2), uWbc_r[...]) + ub1_r[...] + ucomb,
                0.0)
            nf = comb + _mm(uh, uW2_r[...]) + ub2_r[...]
            cols.append(nf)

        store = jnp.concatenate(
            [jnp.zeros((nrows, D_STATE), jnp.float32)] + cols, axis=1)
        sm = store.reshape(PB, NPP, -1) * nvalid
        qs = sm[:, :MAX_SET, :]
        cs = sm[:, MAX_SET:, :]

        def _tr(x):
            h = jnp.maximum(_mm(x.reshape(PB * MAX_SET, D_STATE), t1W_r[...])
                            + t1b_r[...], 0.0)
            return (_mm(h, t2W_r[...]) + t2b_r[...]).reshape(PB, MAX_SET, -1)

        mq = _tr(qs[:, :, -D_STATE:]) * qvalid
        mc = _tr(cs[:, :, -D_STATE:])
        scores = _bmm(mq, mc, 2, 2)

        def _sink(_, la):
            m2 = jnp.max(la, axis=2, keepdims=True)
            la = la - (jnp.log(jnp.sum(jnp.exp(la - m2), axis=2,
                                       keepdims=True)) + m2)
            m1 = jnp.max(la, axis=1, keepdims=True)
            la = la - (jnp.log(jnp.sum(jnp.exp(la - m1), axis=1,
                                       keepdims=True)) + m1)
            return la
        plan = jnp.exp(lax.fori_loop(0, SINK_ITERS, _sink, scores / TEMP))

        if t != N_TIME - 1:
            qfc = _bmm(plan, cs, 2, 1)
            cfq = _bmm(plan, qs, 1, 1)
            ns = jnp.concatenate([qfc, cfq], axis=1)
            store = jnp.concatenate(
                [jnp.zeros((PB, NPP, D_STATE), jnp.float32),
                 ns[:, :, D_STATE:]], axis=2).reshape(nrows, -1)

    out_ref[...] = plan


@jax.jit
def kernel(node_features, edge_features, Wn, bn, We, be, mW1, mb1, mW2, mb2,
           rW1, rb1, rW2, rb2, uW1, ub1, uW2, ub2, cW1, cb1, cW2, cb2,
           t1W, t1b, t2W, t2b, from_idx, to_idx, graph_idx):
    del graph_idx
    nf = node_features.reshape(B_PAIRS, 90, -1)
    nf_pad = jnp.concatenate(
        [nf[:, :QSIZE], jnp.zeros((B_PAIRS, MAX_SET - QSIZE, nf.shape[2]),
                                  nf.dtype), nf[:, QSIZE:]], axis=1)
    ef = edge_features.reshape(B_PAIRS, 180, -1)
    ef_pad = jnp.concatenate(
        [ef[:, :2 * QSIZE], jnp.zeros((B_PAIRS, EPP - 180, ef.shape[2]),
                                      ef.dtype), ef[:, 2 * QSIZE:]], axis=1)

    def _local(idx):
        loc = idx.reshape(B_PAIRS, 180) - 90 * jnp.arange(B_PAIRS,
                                                          dtype=jnp.int32)[:, None]
        q = loc[:, :2 * QSIZE]
        c = loc[:, 2 * QSIZE:] + (MAX_SET - QSIZE)
        pad = jnp.full((B_PAIRS, EPP - 180), PAD_IDX, jnp.int32)
        return jnp.concatenate([q, pad, c], axis=1)

    fl = _local(from_idx)
    tl = _local(to_idx)

    A1, A2, A3 = mW1[:D_STATE], mW1[D_STATE:2 * D_STATE], mW1[2 * D_STATE:]
    rA1, rA2, rA3 = rW1[:D_STATE], rW1[D_STATE:2 * D_STATE], rW1[2 * D_STATE:]
    W_from = jnp.concatenate([A1, rA2], axis=1)
    W_to = jnp.concatenate([A2, rA1], axis=1)
    WeE = We @ jnp.concatenate([A3, rA3], axis=1)
    beE = (be @ jnp.concatenate([A3, rA3], axis=1)
           + jnp.concatenate([mb1, rb1]))[None, :]
    W2m = jax.scipy.linalg.block_diag(mW2, rW2)
    b2m = jnp.concatenate([mb2, rb2])[None, :]
    uW1a, uWbc = uW1[:D_STATE], uW1[D_STATE:]
    cWa = jnp.concatenate([cW2 @ W_from, cW2 @ W_to, cW2 @ uW1a, cW2], axis=1)
    cba = jnp.concatenate([cb2 @ W_from, cb2 @ W_to, cb2 @ uW1a, cb2])[None, :]

    b = lambda v: v.reshape(1, -1)
    weights = [Wn, b(bn), WeE, beE, cW1, b(cb1), cWa, cba,
               W2m, b2m, uWbc, b(ub1), uW2, b(ub2),
               t1W, b(t1b), t2W, b(t2b)]

    grid_spec = pl.GridSpec(
        grid=(GRID,),
        in_specs=[
            pl.BlockSpec((PB, NPP, nf_pad.shape[2]), lambda i: (i, 0, 0)),
            pl.BlockSpec((PB, EPP, ef_pad.shape[2]), lambda i: (i, 0, 0)),
            pl.BlockSpec((PB, EPP), lambda i: (i, 0)),
            pl.BlockSpec((PB, EPP), lambda i: (i, 0)),
        ] + [pl.BlockSpec(w.shape, lambda i: (0, 0)) for w in weights],
        out_specs=pl.BlockSpec((PB, MAX_SET, MAX_SET), lambda i: (i, 0, 0)),
    )
    return pl.pallas_call(
        _body,
        grid_spec=grid_spec,
        out_shape=jax.ShapeDtypeStruct((B_PAIRS, MAX_SET, MAX_SET),
                                       jnp.float32),
        compiler_params=pltpu.CompilerParams(
            dimension_semantics=("parallel",)),
    )(nf_pad, ef_pad, fl, tl, *weights)


# speedup vs baseline: 1.2803x; 1.2803x over previous
import jax
import jax.numpy as jnp
from jax import lax
from jax.experimental import pallas as pl
from jax.experimental.pallas import tpu as pltpu

B_PAIRS = 512
QSIZE = 40
CSIZE = 50
MAX_SET = 50
N_PROP = 5
N_TIME = 3
D_STATE = 32
D_MSG = 64
TEMP = 0.1
SINK_ITERS = 20

PB = 16
GRID = B_PAIRS // PB
NPP = 2 * MAX_SET
EPP = 200
PAD_IDX = 3 * NPP


def _bmm(a, b, ca, cb):
    return lax.dot_general(
        a, b, (((ca,), (cb,)), ((0,), (0,))), preferred_element_type=jnp.float32)


def _mm(x, w):
    return jnp.dot(x, w, preferred_element_type=jnp.float32)


def _gsum(oh, x):
    return _bmm(oh, x, 2, 1)


def _body(nf_ref, ef_ref, fl_ref, tl_ref,
          Wn_r, bn_r, WeE_r, beE_r, cW1_r, cb1_r, cWa_r, cba_r,
          W2m_r, b2m_r, uWbc_r, ub1_r, uW2_r, ub2_r,
          t1W_r, t1b_r, t2W_r, t2b_r,
          out_ref):
    nrows = PB * NPP
    erows = PB * EPP
    H2 = 2 * D_MSG

    enc_n = _mm(nf_ref[...].reshape(nrows, -1), Wn_r[...]) + bn_r[...]
    E3 = (_mm(ef_ref[...].reshape(erows, -1), WeE_r[...])
          + beE_r[...]).reshape(PB, EPP, H2)

    fl = fl_ref[...]
    tl = tl_ref[...]
    i_n = lax.broadcasted_iota(jnp.int32, (PB, EPP, NPP), 2)
    GF = (fl[:, :, None] == i_n).astype(jnp.float32)
    GT = (tl[:, :, None] == i_n).astype(jnp.float32)
    i_r = lax.broadcasted_iota(jnp.int32, (PB, MAX_SET, 2 * MAX_SET), 1)

    def _scat_mat(t_idx, f_idx):
        return jnp.concatenate([
            (t_idx[:, None, :] == i_r).astype(jnp.float32),
            (f_idx[:, None, :] == i_r).astype(jnp.float32)], axis=1)

    Oq = _scat_mat(tl[:, :2 * MAX_SET], fl[:, :2 * MAX_SET])
    Oc = _scat_mat(tl[:, 2 * MAX_SET:] - MAX_SET, fl[:, 2 * MAX_SET:] - MAX_SET)

    i_v = lax.broadcasted_iota(jnp.int32, (1, NPP, 1), 1)
    nvalid = jnp.where((i_v >= QSIZE) & (i_v < MAX_SET), 0.0, 1.0)
    qvalid = (lax.broadcasted_iota(jnp.int32, (1, MAX_SET, 1), 1)
              < QSIZE).astype(jnp.float32)

    store = jnp.zeros((nrows, (N_PROP + 1) * D_STATE), jnp.float32)
    plan = jnp.zeros((PB, MAX_SET, MAX_SET), jnp.float32)

    for t in range(N_TIME):
        nf = enc_n
        cols = []
        for p in range(1, N_PROP + 1):
            inter = store[:, D_STATE * (p - 1):D_STATE * p]
            h1 = jnp.maximum(
                _mm(jnp.concatenate([nf, inter], axis=1), cW1_r[...])
                + cb1_r[...], 0.0)
            CA = _mm(h1, cWa_r[...]) + cba_r[...]
            U = CA[:, :H2].reshape(PB, NPP, H2)
            V = CA[:, H2:2 * H2].reshape(PB, NPP, H2)
            ucomb = CA[:, 2 * H2:2 * H2 + D_MSG]
            comb = CA[:, 2 * H2 + D_MSG:]
            Hh = jnp.maximum(_gsum(GF, U) + _gsum(GT, V) + E3, 0.0)
            X = _mm(Hh.reshape(erows, H2), W2m_r[...]) + b2m_r[...]
            Xp = X.reshape(PB, EPP, H2)
            Rq = _gsum(Oq, Xp[:, :2 * MAX_SET, :])
            Rc = _gsum(Oc, Xp[:, 2 * MAX_SET:, :])
            AR = jnp.concatenate([
                jnp.concatenate([Rq[:, :MAX_SET, :D_MSG],
                                 Rq[:, MAX_SET:, D_MSG:]], axis=2),
                jnp.concatenate([Rc[:, :MAX_SET, :D_MSG],
                                 Rc[:, MAX_SET:, D_MSG:]], axis=2)], axis=1)
            uh = jnp.maximum(
                _mm(AR.reshape(nrows, H2), uWbc_r[...]) + ub1_r[...] + ucomb,
                0.0)
            nf = comb + _mm(uh, uW2_r[...]) + ub2_r[...]
            cols.append(nf)

        store = jnp.concatenate(
            [jnp.zeros((nrows, D_STATE), jnp.float32)] + cols, axis=1)
        sm = store.reshape(PB, NPP, -1) * nvalid
        qs = sm[:, :MAX_SET, :]
        cs = sm[:, MAX_SET:, :]

        def _tr(x):
            h = jnp.maximum(_mm(x.reshape(PB * MAX_SET, D_STATE), t1W_r[...])
                            + t1b_r[...], 0.0)
            return (_mm(h, t2W_r[...]) + t2b_r[...]).reshape(PB, MAX_SET, -1)

        mq = _tr(qs[:, :, -D_STATE:]) * qvalid
        mc = _tr(cs[:, :, -D_STATE:])
        scores = _bmm(mq, mc, 2, 2)

        def _sink(_, la):
            m2 = jnp.max(la, axis=2, keepdims=True)
            la = la - (jnp.log(jnp.sum(jnp.exp(la - m2), axis=2,
                                       keepdims=True)) + m2)
            m1 = jnp.max(la, axis=1, keepdims=True)
            la = la - (jnp.log(jnp.sum(jnp.exp(la - m1), axis=1,
                                       keepdims=True)) + m1)
            return la
        plan = jnp.exp(lax.fori_loop(0, SINK_ITERS, _sink, scores / TEMP))

        if t != N_TIME - 1:
            qfc = _bmm(plan, cs, 2, 1)
            cfq = _bmm(plan, qs, 1, 1)
            ns = jnp.concatenate([qfc, cfq], axis=1)
            store = jnp.concatenate(
                [jnp.zeros((PB, NPP, D_STATE), jnp.float32),
                 ns[:, :, D_STATE:]], axis=2).reshape(nrows, -1)

    out_ref[...] = plan


@jax.jit
def kernel(node_features, edge_features, Wn, bn, We, be, mW1, mb1, mW2, mb2,
           rW1, rb1, rW2, rb2, uW1, ub1, uW2, ub2, cW1, cb1, cW2, cb2,
           t1W, t1b, t2W, t2b, from_idx, to_idx, graph_idx):
    del graph_idx
    nf = node_features.reshape(B_PAIRS, 90, -1)
    nf_pad = jnp.concatenate(
        [nf[:, :QSIZE], jnp.zeros((B_PAIRS, MAX_SET - QSIZE, nf.shape[2]),
                                  nf.dtype), nf[:, QSIZE:]], axis=1)
    ef = edge_features.reshape(B_PAIRS, 180, -1)
    ef_pad = jnp.concatenate(
        [ef[:, :2 * QSIZE], jnp.zeros((B_PAIRS, EPP - 180, ef.shape[2]),
                                      ef.dtype), ef[:, 2 * QSIZE:]], axis=1)

    def _local(idx):
        loc = idx.reshape(B_PAIRS, 180) - 90 * jnp.arange(B_PAIRS,
                                                          dtype=jnp.int32)[:, None]
        q = loc[:, :2 * QSIZE]
        c = loc[:, 2 * QSIZE:] + (MAX_SET - QSIZE)
        pad = jnp.full((B_PAIRS, EPP - 180), PAD_IDX, jnp.int32)
        return jnp.concatenate([q, pad, c], axis=1)

    fl = _local(from_idx)
    tl = _local(to_idx)

    A1, A2, A3 = mW1[:D_STATE], mW1[D_STATE:2 * D_STATE], mW1[2 * D_STATE:]
    rA1, rA2, rA3 = rW1[:D_STATE], rW1[D_STATE:2 * D_STATE], rW1[2 * D_STATE:]
    W_from = jnp.concatenate([A1, rA2], axis=1)
    W_to = jnp.concatenate([A2, rA1], axis=1)
    WeE = We @ jnp.concatenate([A3, rA3], axis=1)
    beE = (be @ jnp.concatenate([A3, rA3], axis=1)
           + jnp.concatenate([mb1, rb1]))[None, :]
    W2m = jax.scipy.linalg.block_diag(mW2, rW2)
    b2m = jnp.concatenate([mb2, rb2])[None, :]
    uW1a, uWbc = uW1[:D_STATE], uW1[D_STATE:]
    cWa = jnp.concatenate([cW2 @ W_from, cW2 @ W_to, cW2 @ uW1a, cW2], axis=1)
    cba = jnp.concatenate([cb2 @ W_from, cb2 @ W_to, cb2 @ uW1a, cb2])[None, :]

    b = lambda v: v.reshape(1, -1)
    weights = [Wn, b(bn), WeE, beE, cW1, b(cb1), cWa, cba,
               W2m, b2m, uWbc, b(ub1), uW2, b(ub2),
               t1W, b(t1b), t2W, b(t2b)]

    grid_spec = pl.GridSpec(
        grid=(GRID,),
        in_specs=[
            pl.BlockSpec((PB, NPP, nf_pad.shape[2]), lambda i: (i, 0, 0)),
            pl.BlockSpec((PB, EPP, ef_pad.shape[2]), lambda i: (i, 0, 0)),
            pl.BlockSpec((PB, EPP), lambda i: (i, 0)),
            pl.BlockSpec((PB, EPP), lambda i: (i, 0)),
        ] + [pl.BlockSpec(w.shape, lambda i: (0, 0)) for w in weights],
        out_specs=pl.BlockSpec((PB, MAX_SET, MAX_SET), lambda i: (i, 0, 0)),
    )
    return pl.pallas_call(
        _body,
        grid_spec=grid_spec,
        out_shape=jax.ShapeDtypeStruct((B_PAIRS, MAX_SET, MAX_SET),
                                       jnp.float32),
        compiler_params=pltpu.CompilerParams(
            dimension_semantics=("parallel",)),
    )(nf_pad, ef_pad, fl, tl, *weights)
